# Initial kernel scaffold; baseline (speedup 1.0000x reference)
import jax
import jax.numpy as jnp
from jax.experimental import pallas as pl


def kernel(user_segment, item_embeddings, context_embedding, W_ih, W_hh, b_ih, b_hh, attn_W, attn_b, ue_W, ue_b, ce_W, ce_b, fus_W, fus_b, cs_W1, cs_b1, cs_W2, cs_b2):
    raise NotImplementedError("write your pallas kernel here")


# speedup vs baseline: 1.2432x; 1.2432x over previous
import jax
import jax.numpy as jnp
from jax.experimental import pallas as pl
from jax.experimental.pallas import tpu as pltpu

B, L, D, C, H = 4096, 50, 128, 128, 128
R = 256
G = B // R


def _encoder_block(seg_ref, x_ref, ctx_ref,
                   W_ih_ref, W_hh_ref, b_ih_ref, b_hh_ref,
                   attn_W_ref, attn_b_ref,
                   ue_W_ref, ue_b_ref, ce_W_ref, ce_b_ref,
                   fus_W_ref, fus_b_ref,
                   cs_W1_ref, cs_b1_ref, cs_W2_ref, cs_b2_ref,
                   out_ref):
    f32 = jnp.float32
    W_ih = W_ih_ref[...]
    W_hh = W_hh_ref[...]
    b_ih = b_ih_ref[...]
    b_hh = b_hh_ref[...]
    attn_W = attn_W_ref[...]
    attn_b = attn_b_ref[...]

    h = jnp.zeros((R, H), f32)
    m = jnp.full((R, 1), -1e30, f32)
    s = jnp.zeros((R, 1), f32)
    acc = jnp.zeros((R, H), f32)
    xsum = jnp.zeros((R, D), f32)

    for t in range(L):
        xt = x_ref[:, t * D:(t + 1) * D]
        xsum = xsum + xt
        gi = jnp.dot(xt, W_ih, preferred_element_type=f32) + b_ih
        gh = jnp.dot(h, W_hh, preferred_element_type=f32) + b_hh
        r = jax.nn.sigmoid(gi[:, :H] + gh[:, :H])
        z = jax.nn.sigmoid(gi[:, H:2 * H] + gh[:, H:2 * H])
        n = jnp.tanh(gi[:, 2 * H:] + r * gh[:, 2 * H:])
        h = (1.0 - z) * n + z * h
        sc = jnp.dot(h, attn_W, preferred_element_type=f32) + attn_b
        m2 = jnp.maximum(m, sc)
        alpha = jnp.exp(m - m2)
        p = jnp.exp(sc - m2)
        s = s * alpha + p
        acc = acc * alpha + p * h
        m = m2

    power = acc / s

    ctx = ctx_ref[...]
    mean = xsum * (1.0 / L)
    ue = jax.nn.relu(jnp.dot(mean, ue_W_ref[...], preferred_element_type=f32)
                     + ue_b_ref[...])
    ce = jax.nn.relu(jnp.dot(ctx, ce_W_ref[...], preferred_element_type=f32)
                     + ce_b_ref[...])
    fus_W = fus_W_ref[...]
    reg = (jnp.dot(ue, fus_W[:H], preferred_element_type=f32)
           + jnp.dot(ce, fus_W[H:], preferred_element_type=f32)
           + fus_b_ref[...])
    cold = (jnp.dot(jax.nn.relu(jnp.dot(ctx, cs_W1_ref[...],
                                        preferred_element_type=f32)
                                + cs_b1_ref[...]),
                    cs_W2_ref[...], preferred_element_type=f32)
            + cs_b2_ref[...])

    seg = seg_ref[...]
    out_ref[...] = jnp.where(seg == 0, power, jnp.where(seg == 1, reg, cold))


def kernel(user_segment, item_embeddings, context_embedding,
           W_ih, W_hh, b_ih, b_hh, attn_W, attn_b,
           ue_W, ue_b, ce_W, ce_b, fus_W, fus_b,
           cs_W1, cs_b1, cs_W2, cs_b2):
    x2 = item_embeddings.reshape(B, L * D)
    seg2 = user_segment.reshape(B, 1)

    def full(shape):
        return pl.BlockSpec(shape, lambda b: (0,) * len(shape))

    grid_spec = pl.GridSpec(
        grid=(G,),
        in_specs=[
            pl.BlockSpec((R, 1), lambda b: (b, 0)),
            pl.BlockSpec((R, L * D), lambda b: (b, 0)),
            pl.BlockSpec((R, C), lambda b: (b, 0)),
            full((D, 3 * H)), full((H, 3 * H)),
            full((1, 3 * H)), full((1, 3 * H)),
            full((H, 1)), full((1, 1)),
            full((D, H)), full((1, H)),
            full((C, H)), full((1, H)),
            full((2 * H, H)), full((1, H)),
            full((C, H)), full((1, H)),
            full((H, H)), full((1, H)),
        ],
        out_specs=pl.BlockSpec((R, H), lambda b: (b, 0)),
    )

    return pl.pallas_call(
        _encoder_block,
        grid_spec=grid_spec,
        out_shape=jax.ShapeDtypeStruct((B, H), jnp.float32),
    )(seg2, x2, context_embedding,
      W_ih, W_hh, b_ih.reshape(1, 3 * H), b_hh.reshape(1, 3 * H),
      attn_W, attn_b.reshape(1, 1),
      ue_W, ue_b.reshape(1, H), ce_W, ce_b.reshape(1, H),
      fus_W, fus_b.reshape(1, H),
      cs_W1, cs_b1.reshape(1, H), cs_W2, cs_b2.reshape(1, H))


# speedup vs baseline: 1.4021x; 1.1278x over previous
import functools

import jax
import jax.numpy as jnp
from jax import lax
from jax.experimental import pallas as pl
from jax.experimental.pallas import tpu as pltpu
from jax.experimental.pallas import tpu_sc as plsc

B, L, D, C, H = 4096, 50, 128, 128, 128
R = 256
G = B // R
NW = 32
RW = B // NW

_sc_mesh = plsc.VectorSubcoreMesh(core_axis_name="c", subcore_axis_name="s")
_sc_params = pltpu.CompilerParams(needs_layout_passes=False)


@functools.partial(
    pl.kernel,
    out_type=[jax.ShapeDtypeStruct((B,), jnp.int32),
              jax.ShapeDtypeStruct((16,), jnp.int32)],
    mesh=_sc_mesh,
    compiler_params=_sc_params,
    scratch_types=[pltpu.VMEM((B,), jnp.int32),
                   pltpu.VMEM((B,), jnp.int32),
                   pltpu.VMEM((16,), jnp.int32)],
)
def _route_sc(seg_hbm, perm_hbm, cnt_hbm, seg_v, perm_v, cnt_v):
    wid = lax.axis_index("s") * 2 + lax.axis_index("c")

    @pl.when(wid == 0)
    def _():
        pltpu.sync_copy(seg_hbm, seg_v)
        lanes = lax.iota(jnp.int32, 16)

        one = jnp.ones((16,), jnp.int32)
        nil = jnp.zeros((16,), jnp.int32)

        def count_body(i, carry):
            c0, c1 = carry
            v = seg_v[pl.ds(i * 16, 16)]
            c0 = c0 + jnp.sum(jnp.where(v == 0, one, nil))
            c1 = c1 + jnp.sum(jnp.where(v == 1, one, nil))
            return c0, c1

        c0, c1 = lax.fori_loop(0, B // 16, count_body,
                               (jnp.int32(0), jnp.int32(0)))

        def scat_body(i, bases):
            b0, b1, b2 = bases
            v = seg_v[pl.ds(i * 16, 16)]
            rowid = lanes + i * 16
            m0 = v == 0
            m1 = v == 1
            m2 = v == 2
            p0 = plsc.cumsum(jnp.where(m0, one, nil))
            p1 = plsc.cumsum(jnp.where(m1, one, nil))
            p2 = plsc.cumsum(jnp.where(m2, one, nil))
            plsc.store_scatter(perm_v, [b0 + p0 - 1], rowid, mask=m0)
            plsc.store_scatter(perm_v, [b1 + p1 - 1], rowid, mask=m1)
            plsc.store_scatter(perm_v, [b2 + p2 - 1], rowid, mask=m2)
            return (b0 + jnp.sum(jnp.where(m0, one, nil)),
                    b1 + jnp.sum(jnp.where(m1, one, nil)),
                    b2 + jnp.sum(jnp.where(m2, one, nil)))

        lax.fori_loop(0, B // 16, scat_body, (jnp.int32(0), c0, c0 + c1))

        cnt_v[...] = jnp.where(lanes == 0, c0,
                               jnp.where(lanes == 1, c0 + c1, nil))
        pltpu.sync_copy(perm_v, perm_hbm)
        pltpu.sync_copy(cnt_v, cnt_hbm)


@functools.partial(
    pl.kernel,
    out_type=jax.ShapeDtypeStruct((B, H), jnp.float32),
    mesh=_sc_mesh,
    compiler_params=_sc_params,
    scratch_types=[pltpu.VMEM((RW,), jnp.int32),
                   pltpu.VMEM((RW, H), jnp.float32),
                   pltpu.SemaphoreType.DMA],
)
def _scatter_sc(perm_hbm, outc_hbm, out_hbm, idx_v, rows_v, sem):
    wid = lax.axis_index("s") * 2 + lax.axis_index("c")
    base = wid * RW
    pltpu.sync_copy(perm_hbm.at[pl.ds(base, RW)], idx_v)
    pltpu.sync_copy(outc_hbm.at[pl.ds(base, RW)], rows_v)
    pltpu.async_copy(rows_v, out_hbm.at[idx_v], sem).wait()


def _experts_tc(perm_sref, cnt_sref, item_hbm, ctx_hbm,
                W_ih_ref, W_hh_ref, b_ih_ref, b_hh_ref,
                attn_W_ref, attn_b_ref,
                ue_W_ref, ue_b_ref, ce_W_ref, ce_b_ref,
                fus_W_ref, fus_b_ref,
                cs_W1_ref, cs_b1_ref, cs_W2_ref, cs_b2_ref,
                out_ref, xacc, ctxacc, isem, csem):
    f32 = jnp.float32
    start = pl.program_id(0) * R
    c0 = cnt_sref[0]
    c01 = cnt_sref[1]

    def cissue(r, carry):
        idx = perm_sref[start + r]
        pltpu.make_async_copy(ctx_hbm.at[idx], ctxacc.at[r], csem).start()
        return carry

    lax.fori_loop(0, R, cissue, 0)

    @pl.when(start < c01)
    def _():
        def iissue(r, carry):
            idx = perm_sref[jnp.minimum(start + r, c01 - 1)]
            pltpu.make_async_copy(item_hbm.at[idx], xacc.at[r], isem).start()
            return carry

        lax.fori_loop(0, R, iissue, 0)

        def idrain(r, carry):
            pltpu.make_async_copy(item_hbm.at[0], xacc.at[0], isem).wait()
            return carry

        lax.fori_loop(0, R, idrain, 0)

    def cdrain(r, carry):
        pltpu.make_async_copy(ctx_hbm.at[0], ctxacc.at[0], csem).wait()
        return carry

    lax.fori_loop(0, R, cdrain, 0)

    rowpos = start + lax.broadcasted_iota(jnp.int32, (R, 1), 0)
    ctx = ctxacc[...]

    cold = (jnp.dot(jax.nn.relu(jnp.dot(ctx, cs_W1_ref[...],
                                        preferred_element_type=f32)
                                + cs_b1_ref[...]),
                    cs_W2_ref[...], preferred_element_type=f32)
            + cs_b2_ref[...])
    out_ref[...] = cold

    @pl.when(jnp.logical_and(start < c01, start + R > c0))
    def _():
        xsum = jnp.zeros((R, D), f32)
        for t in range(L):
            xsum = xsum + xacc[:, t * D:(t + 1) * D]
        mean = xsum * (1.0 / L)
        ue = jax.nn.relu(jnp.dot(mean, ue_W_ref[...],
                                 preferred_element_type=f32) + ue_b_ref[...])
        ce = jax.nn.relu(jnp.dot(ctx, ce_W_ref[...],
                                 preferred_element_type=f32) + ce_b_ref[...])
        fus_W = fus_W_ref[...]
        reg = (jnp.dot(ue, fus_W[:H], preferred_element_type=f32)
               + jnp.dot(ce, fus_W[H:], preferred_element_type=f32)
               + fus_b_ref[...])
        keep = jnp.logical_and(rowpos >= c0, rowpos < c01)
        out_ref[...] = jnp.where(keep, reg, out_ref[...])

    @pl.when(start < c0)
    def _():
        W_ih = W_ih_ref[...]
        W_hh = W_hh_ref[...]
        b_ih = b_ih_ref[...]
        b_hh = b_hh_ref[...]
        attn_W = attn_W_ref[...]
        attn_b = attn_b_ref[...]

        h = jnp.zeros((R, H), f32)
        m = jnp.full((R, 1), -1e30, f32)
        s = jnp.zeros((R, 1), f32)
        acc = jnp.zeros((R, H), f32)
        for t in range(L):
            xt = xacc[:, t * D:(t + 1) * D]
            gi = jnp.dot(xt, W_ih, preferred_element_type=f32) + b_ih
            gh = jnp.dot(h, W_hh, preferred_element_type=f32) + b_hh
            rg = jax.nn.sigmoid(gi[:, :H] + gh[:, :H])
            z = jax.nn.sigmoid(gi[:, H:2 * H] + gh[:, H:2 * H])
            n = jnp.tanh(gi[:, 2 * H:] + rg * gh[:, 2 * H:])
            h = (1.0 - z) * n + z * h
            sc = jnp.dot(h, attn_W, preferred_element_type=f32) + attn_b
            m2 = jnp.maximum(m, sc)
            alpha = jnp.exp(m - m2)
            p = jnp.exp(sc - m2)
            s = s * alpha + p
            acc = acc * alpha + p * h
            m = m2
        power = acc / s
        out_ref[...] = jnp.where(rowpos < c0, power, out_ref[...])


def _experts_call(perm, cnt, x2, ctx, *weights):
    def full(shape):
        return pl.BlockSpec(shape, lambda b, *_: (0,) * len(shape))

    grid_spec = pltpu.PrefetchScalarGridSpec(
        num_scalar_prefetch=2,
        grid=(G,),
        in_specs=[
            pl.BlockSpec(memory_space=pltpu.MemorySpace.HBM),
            pl.BlockSpec(memory_space=pltpu.MemorySpace.HBM),
            full((D, 3 * H)), full((H, 3 * H)),
            full((1, 3 * H)), full((1, 3 * H)),
            full((H, 1)), full((1, 1)),
            full((D, H)), full((1, H)),
            full((C, H)), full((1, H)),
            full((2 * H, H)), full((1, H)),
            full((C, H)), full((1, H)),
            full((H, H)), full((1, H)),
        ],
        out_specs=pl.BlockSpec((R, H), lambda b, *_: (b, 0)),
        scratch_shapes=[
            pltpu.VMEM((R, L * D), jnp.float32),
            pltpu.VMEM((R, C), jnp.float32),
            pltpu.SemaphoreType.DMA,
            pltpu.SemaphoreType.DMA,
        ],
    )
    return pl.pallas_call(
        _experts_tc,
        grid_spec=grid_spec,
        out_shape=jax.ShapeDtypeStruct((B, H), jnp.float32),
    )(perm, cnt, x2, ctx, *weights)


def kernel(user_segment, item_embeddings, context_embedding,
           W_ih, W_hh, b_ih, b_hh, attn_W, attn_b,
           ue_W, ue_b, ce_W, ce_b, fus_W, fus_b,
           cs_W1, cs_b1, cs_W2, cs_b2):
    x2 = item_embeddings.reshape(B, L * D)
    seg = user_segment.astype(jnp.int32)

    perm, cnt = _route_sc(seg)
    out_c = _experts_call(
        perm, cnt, x2, context_embedding,
        W_ih, W_hh, b_ih.reshape(1, 3 * H), b_hh.reshape(1, 3 * H),
        attn_W, attn_b.reshape(1, 1),
        ue_W, ue_b.reshape(1, H), ce_W, ce_b.reshape(1, H),
        fus_W, fus_b.reshape(1, H),
        cs_W1, cs_b1.reshape(1, H), cs_W2, cs_b2.reshape(1, H))
    return _scatter_sc(perm, out_c)


# speedup vs baseline: 2.2514x; 1.6057x over previous
import functools

import jax
import jax.numpy as jnp
from jax import lax
from jax.experimental import pallas as pl
from jax.experimental.pallas import tpu as pltpu
from jax.experimental.pallas import tpu_sc as plsc

B, L, D, C, H = 4096, 50, 128, 128, 128
R = 256
G = B // R
NW = 32
RW = B // NW

_sc_mesh = plsc.VectorSubcoreMesh(core_axis_name="c", subcore_axis_name="s")
_sc_params = pltpu.CompilerParams(needs_layout_passes=False)


@functools.partial(
    pl.kernel,
    out_type=[jax.ShapeDtypeStruct((B,), jnp.int32),
              jax.ShapeDtypeStruct((16,), jnp.int32)],
    mesh=_sc_mesh,
    compiler_params=_sc_params,
    scratch_types=[pltpu.VMEM((B,), jnp.int32),
                   pltpu.VMEM((B,), jnp.int32),
                   pltpu.VMEM((16,), jnp.int32)],
)
def _route_sc(seg_hbm, perm_hbm, cnt_hbm, seg_v, perm_v, cnt_v):
    wid = lax.axis_index("s") * 2 + lax.axis_index("c")

    @pl.when(wid == 0)
    def _():
        pltpu.sync_copy(seg_hbm, seg_v)
        lanes = lax.iota(jnp.int32, 16)
        one = jnp.ones((16,), jnp.int32)
        nil = jnp.zeros((16,), jnp.int32)

        def count_body(i, carry):
            c0, c1 = carry
            v = seg_v[pl.ds(i * 16, 16)]
            c0 = c0 + jnp.sum(jnp.where(v == 0, one, nil))
            c1 = c1 + jnp.sum(jnp.where(v == 1, one, nil))
            return c0, c1

        c0, c1 = lax.fori_loop(0, B // 16, count_body,
                               (jnp.int32(0), jnp.int32(0)))

        def scat_body(i, bases):
            b0, b1, b2 = bases
            v = seg_v[pl.ds(i * 16, 16)]
            rowid = lanes + i * 16
            m0 = v == 0
            m1 = v == 1
            m2 = v == 2
            p0 = plsc.cumsum(jnp.where(m0, one, nil))
            p1 = plsc.cumsum(jnp.where(m1, one, nil))
            p2 = plsc.cumsum(jnp.where(m2, one, nil))
            plsc.store_scatter(perm_v, [b0 + p0 - 1], rowid, mask=m0)
            plsc.store_scatter(perm_v, [b1 + p1 - 1], rowid, mask=m1)
            plsc.store_scatter(perm_v, [b2 + p2 - 1], rowid, mask=m2)
            return (b0 + jnp.sum(jnp.where(m0, one, nil)),
                    b1 + jnp.sum(jnp.where(m1, one, nil)),
                    b2 + jnp.sum(jnp.where(m2, one, nil)))

        lax.fori_loop(0, B // 16, scat_body, (jnp.int32(0), c0, c0 + c1))

        cnt_v[...] = jnp.where(lanes == 0, c0,
                               jnp.where(lanes == 1, c0 + c1, nil))
        pltpu.sync_copy(perm_v, perm_hbm)
        pltpu.sync_copy(cnt_v, cnt_hbm)


@functools.partial(
    pl.kernel,
    out_type=jax.ShapeDtypeStruct((B, C), jnp.float32),
    mesh=_sc_mesh,
    compiler_params=_sc_params,
    scratch_types=[pltpu.VMEM((RW,), jnp.int32),
                   pltpu.VMEM((RW, C), jnp.float32),
                   pltpu.SemaphoreType.DMA],
)
def _gather_ctx_sc(perm_hbm, ctx_hbm, ctxc_hbm, idx_v, rows_v, sem):
    wid = lax.axis_index("s") * 2 + lax.axis_index("c")
    base = wid * RW
    pltpu.sync_copy(perm_hbm.at[pl.ds(base, RW)], idx_v)
    pltpu.async_copy(ctx_hbm.at[idx_v], rows_v, sem).wait()
    pltpu.sync_copy(rows_v, ctxc_hbm.at[pl.ds(base, RW)])


@functools.partial(
    pl.kernel,
    out_type=jax.ShapeDtypeStruct((B, H), jnp.float32),
    mesh=_sc_mesh,
    compiler_params=_sc_params,
    scratch_types=[pltpu.VMEM((RW,), jnp.int32),
                   pltpu.VMEM((RW, H), jnp.float32),
                   pltpu.SemaphoreType.DMA],
)
def _scatter_sc(perm_hbm, outc_hbm, out_hbm, idx_v, rows_v, sem):
    wid = lax.axis_index("s") * 2 + lax.axis_index("c")
    base = wid * RW
    pltpu.sync_copy(perm_hbm.at[pl.ds(base, RW)], idx_v)
    pltpu.sync_copy(outc_hbm.at[pl.ds(base, RW)], rows_v)
    pltpu.async_copy(rows_v, out_hbm.at[idx_v], sem).wait()


def _experts_tc(perm_sref, cnt_sref, item_hbm, ctxc_ref,
                W_ih_ref, W_hh_ref, b_ih_ref, b_hh_ref,
                attn_W_ref, attn_b_ref,
                ue_W_ref, ue_b_ref, ce_W_ref, ce_b_ref,
                fus_W_ref, fus_b_ref,
                cs_W1_ref, cs_b1_ref, cs_W2_ref, cs_b2_ref,
                out_ref, xacc, sem):
    f32 = jnp.float32
    b = pl.program_id(0)
    start = b * R
    c0 = cnt_sref[0]
    c01 = cnt_sref[1]
    par = lax.rem(b, 2)

    def issue(blk, buf):
        s0 = blk * R

        def body(r, carry):
            idx = perm_sref[jnp.minimum(s0 + r, c01 - 1)]
            pltpu.make_async_copy(item_hbm.at[idx], xacc.at[buf, r],
                                  sem.at[buf]).start()
            return carry

        lax.fori_loop(0, R, body, 0, unroll=8)

    @pl.when(jnp.logical_and(b == 0, c01 > 0))
    def _():
        issue(0, 0)

    @pl.when(jnp.logical_and(b + 1 < G, (b + 1) * R < c01))
    def _():
        issue(b + 1, 1 - par)

    @pl.when(start < c01)
    def _():
        pltpu.make_async_copy(item_hbm.at[pl.ds(0, R)], xacc.at[par],
                              sem.at[par]).wait()

    xp = xacc.at[par]
    rowpos = start + lax.broadcasted_iota(jnp.int32, (R, 1), 0)
    ctx = ctxc_ref[...]

    cold = (jnp.dot(jax.nn.relu(jnp.dot(ctx, cs_W1_ref[...],
                                        preferred_element_type=f32)
                                + cs_b1_ref[...]),
                    cs_W2_ref[...], preferred_element_type=f32)
            + cs_b2_ref[...])
    out_ref[...] = cold

    @pl.when(jnp.logical_and(start < c01, start + R > c0))
    def _():
        xsum = jnp.zeros((R, D), f32)
        for t in range(L):
            xsum = xsum + xp[:, t, :]
        mean = xsum * (1.0 / L)
        ue = jax.nn.relu(jnp.dot(mean, ue_W_ref[...],
                                 preferred_element_type=f32) + ue_b_ref[...])
        ce = jax.nn.relu(jnp.dot(ctx, ce_W_ref[...],
                                 preferred_element_type=f32) + ce_b_ref[...])
        fus_W = fus_W_ref[...]
        reg = (jnp.dot(ue, fus_W[:H], preferred_element_type=f32)
               + jnp.dot(ce, fus_W[H:], preferred_element_type=f32)
               + fus_b_ref[...])
        keep = jnp.logical_and(rowpos >= c0, rowpos < c01)
        out_ref[...] = jnp.where(keep, reg, out_ref[...])

    @pl.when(start < c0)
    def _():
        W_ih = W_ih_ref[...]
        W_hh = W_hh_ref[...]
        b_ih = b_ih_ref[...]
        b_hh = b_hh_ref[...]
        attn_W = attn_W_ref[...]
        attn_b = attn_b_ref[...]

        h = jnp.zeros((R, H), f32)
        m = jnp.full((R, 1), -1e30, f32)
        s = jnp.zeros((R, 1), f32)
        acc = jnp.zeros((R, H), f32)
        for t in range(L):
            xt = xp[:, t, :]
            gi = jnp.dot(xt, W_ih, preferred_element_type=f32) + b_ih
            gh = jnp.dot(h, W_hh, preferred_element_type=f32) + b_hh
            rg = jax.nn.sigmoid(gi[:, :H] + gh[:, :H])
            z = jax.nn.sigmoid(gi[:, H:2 * H] + gh[:, H:2 * H])
            n = jnp.tanh(gi[:, 2 * H:] + rg * gh[:, 2 * H:])
            h = (1.0 - z) * n + z * h
            sc = jnp.dot(h, attn_W, preferred_element_type=f32) + attn_b
            m2 = jnp.maximum(m, sc)
            alpha = jnp.exp(m - m2)
            p = jnp.exp(sc - m2)
            s = s * alpha + p
            acc = acc * alpha + p * h
            m = m2
        power = acc / s
        out_ref[...] = jnp.where(rowpos < c0, power, out_ref[...])


def _experts_call(perm, cnt, item, ctx_c, *weights):
    def full(shape):
        return pl.BlockSpec(shape, lambda b, *_: (0,) * len(shape))

    grid_spec = pltpu.PrefetchScalarGridSpec(
        num_scalar_prefetch=2,
        grid=(G,),
        in_specs=[
            pl.BlockSpec(memory_space=pltpu.MemorySpace.HBM),
            pl.BlockSpec((R, C), lambda b, *_: (b, 0)),
            full((D, 3 * H)), full((H, 3 * H)),
            full((1, 3 * H)), full((1, 3 * H)),
            full((H, 1)), full((1, 1)),
            full((D, H)), full((1, H)),
            full((C, H)), full((1, H)),
            full((2 * H, H)), full((1, H)),
            full((C, H)), full((1, H)),
            full((H, H)), full((1, H)),
        ],
        out_specs=pl.BlockSpec((R, H), lambda b, *_: (b, 0)),
        scratch_shapes=[
            pltpu.VMEM((2, R, L, D), jnp.float32),
            pltpu.SemaphoreType.DMA((2,)),
        ],
    )
    return pl.pallas_call(
        _experts_tc,
        grid_spec=grid_spec,
        out_shape=jax.ShapeDtypeStruct((B, H), jnp.float32),
    )(perm, cnt, item, ctx_c, *weights)


def kernel(user_segment, item_embeddings, context_embedding,
           W_ih, W_hh, b_ih, b_hh, attn_W, attn_b,
           ue_W, ue_b, ce_W, ce_b, fus_W, fus_b,
           cs_W1, cs_b1, cs_W2, cs_b2):
    seg = user_segment.astype(jnp.int32)

    perm, cnt = _route_sc(seg)
    ctx_c = _gather_ctx_sc(perm, context_embedding)
    out_c = _experts_call(
        perm, cnt, item_embeddings, ctx_c,
        W_ih, W_hh, b_ih.reshape(1, 3 * H), b_hh.reshape(1, 3 * H),
        attn_W, attn_b.reshape(1, 1),
        ue_W, ue_b.reshape(1, H), ce_W, ce_b.reshape(1, H),
        fus_W, fus_b.reshape(1, H),
        cs_W1, cs_b1.reshape(1, H), cs_W2, cs_b2.reshape(1, H))
    return _scatter_sc(perm, out_c)


# speedup vs baseline: 2.4686x; 1.0965x over previous
import functools

import jax
import jax.numpy as jnp
from jax import lax
from jax.experimental import pallas as pl
from jax.experimental.pallas import tpu as pltpu
from jax.experimental.pallas import tpu_sc as plsc

B, L, D, C, H = 4096, 50, 128, 128, 128
R = 256
G = B // R
NW = 32
RW = B // NW

_sc_mesh = plsc.VectorSubcoreMesh(core_axis_name="c", subcore_axis_name="s")
_sc_params = pltpu.CompilerParams(needs_layout_passes=False)


@functools.partial(
    pl.kernel,
    out_type=[jax.ShapeDtypeStruct((B,), jnp.int32),
              jax.ShapeDtypeStruct((16,), jnp.int32)],
    mesh=_sc_mesh,
    compiler_params=_sc_params,
    scratch_types=[pltpu.VMEM((B,), jnp.int32),
                   pltpu.VMEM((B,), jnp.int32),
                   pltpu.VMEM((16,), jnp.int32)],
)
def _route_sc(seg_hbm, perm_hbm, cnt_hbm, seg_v, perm_v, cnt_v):
    wid = lax.axis_index("s") * 2 + lax.axis_index("c")

    @pl.when(wid == 0)
    def _():
        pltpu.sync_copy(seg_hbm, seg_v)
        lanes = lax.iota(jnp.int32, 16)
        one = jnp.ones((16,), jnp.int32)
        nil = jnp.zeros((16,), jnp.int32)

        def count_body(i, carry):
            c0, c1 = carry
            v = seg_v[pl.ds(i * 16, 16)]
            c0 = c0 + jnp.sum(jnp.where(v == 0, one, nil))
            c1 = c1 + jnp.sum(jnp.where(v == 1, one, nil))
            return c0, c1

        c0, c1 = lax.fori_loop(0, B // 16, count_body,
                               (jnp.int32(0), jnp.int32(0)))

        def scat_body(i, bases):
            b0, b1, b2 = bases
            v = seg_v[pl.ds(i * 16, 16)]
            rowid = lanes + i * 16
            m0 = v == 0
            m1 = v == 1
            m2 = v == 2
            p0 = plsc.cumsum(jnp.where(m0, one, nil))
            p1 = plsc.cumsum(jnp.where(m1, one, nil))
            p2 = plsc.cumsum(jnp.where(m2, one, nil))
            plsc.store_scatter(perm_v, [b0 + p0 - 1], rowid, mask=m0)
            plsc.store_scatter(perm_v, [b1 + p1 - 1], rowid, mask=m1)
            plsc.store_scatter(perm_v, [b2 + p2 - 1], rowid, mask=m2)
            return (b0 + jnp.sum(jnp.where(m0, one, nil)),
                    b1 + jnp.sum(jnp.where(m1, one, nil)),
                    b2 + jnp.sum(jnp.where(m2, one, nil)))

        lax.fori_loop(0, B // 16, scat_body, (jnp.int32(0), c0, c0 + c1))

        cnt_v[...] = jnp.where(lanes == 0, c0,
                               jnp.where(lanes == 1, c0 + c1, nil))
        pltpu.sync_copy(perm_v, perm_hbm)
        pltpu.sync_copy(cnt_v, cnt_hbm)


@functools.partial(
    pl.kernel,
    out_type=jax.ShapeDtypeStruct((B, C), jnp.float32),
    mesh=_sc_mesh,
    compiler_params=_sc_params,
    scratch_types=[pltpu.VMEM((RW,), jnp.int32),
                   pltpu.VMEM((RW, C), jnp.float32),
                   pltpu.SemaphoreType.DMA],
)
def _gather_ctx_sc(perm_hbm, ctx_hbm, ctxc_hbm, idx_v, rows_v, sem):
    wid = lax.axis_index("s") * 2 + lax.axis_index("c")
    base = wid * RW
    pltpu.sync_copy(perm_hbm.at[pl.ds(base, RW)], idx_v)
    pltpu.async_copy(ctx_hbm.at[idx_v], rows_v, sem).wait()
    pltpu.sync_copy(rows_v, ctxc_hbm.at[pl.ds(base, RW)])


@functools.partial(
    pl.kernel,
    out_type=jax.ShapeDtypeStruct((B, H), jnp.float32),
    mesh=_sc_mesh,
    compiler_params=_sc_params,
    scratch_types=[pltpu.VMEM((RW,), jnp.int32),
                   pltpu.VMEM((RW, H), jnp.float32),
                   pltpu.SemaphoreType.DMA],
)
def _scatter_sc(perm_hbm, outc_hbm, out_hbm, idx_v, rows_v, sem):
    wid = lax.axis_index("s") * 2 + lax.axis_index("c")
    base = wid * RW
    pltpu.sync_copy(perm_hbm.at[pl.ds(base, RW)], idx_v)
    pltpu.sync_copy(outc_hbm.at[pl.ds(base, RW)], rows_v)
    pltpu.async_copy(rows_v, out_hbm.at[idx_v], sem).wait()


def _experts_tc(perm_sref, cnt_sref, item_hbm, ctxc_ref,
                W_ih_ref, W_hh_ref, b_ih_ref, b_hh_ref,
                attn_W_ref, attn_b_ref,
                ue_W_ref, ue_b_ref, ce_W_ref, ce_b_ref,
                fus_W_ref, fus_b_ref,
                cs_W1_ref, cs_b1_ref, cs_W2_ref, cs_b2_ref,
                out_ref, xacc, sem):
    f32 = jnp.float32
    b = pl.program_id(0)
    start = b * R
    c0 = cnt_sref[0]
    c01 = cnt_sref[1]
    par = lax.rem(b, 2)

    def issue(blk, buf):
        s0 = blk * R

        def body(r, carry):
            idx = perm_sref[jnp.minimum(s0 + r, c01 - 1)]
            pltpu.make_async_copy(item_hbm.at[idx], xacc.at[buf, :, r, :],
                                  sem.at[buf]).start()
            return carry

        lax.fori_loop(0, R, body, 0, unroll=8)

    @pl.when(jnp.logical_and(b == 0, c01 > 0))
    def _():
        issue(0, 0)

    @pl.when(jnp.logical_and(b + 1 < G, (b + 1) * R < c01))
    def _():
        issue(b + 1, 1 - par)

    @pl.when(start < c01)
    def _():
        def dbody(r, carry):
            pltpu.make_async_copy(item_hbm.at[0], xacc.at[par, :, 0, :],
                                  sem.at[par]).wait()
            return carry

        lax.fori_loop(0, R, dbody, 0, unroll=8)

    xp = xacc.at[par]
    rowpos = start + lax.broadcasted_iota(jnp.int32, (R, 1), 0)
    ctx = ctxc_ref[...]

    cold = (jnp.dot(jax.nn.relu(jnp.dot(ctx, cs_W1_ref[...],
                                        preferred_element_type=f32)
                                + cs_b1_ref[...]),
                    cs_W2_ref[...], preferred_element_type=f32)
            + cs_b2_ref[...])
    out_ref[...] = cold

    @pl.when(jnp.logical_and(start < c01, start + R > c0))
    def _():
        xsum = jnp.zeros((R, D), f32)
        for t in range(L):
            xsum = xsum + xp[t]
        mean = xsum * (1.0 / L)
        ue = jax.nn.relu(jnp.dot(mean, ue_W_ref[...],
                                 preferred_element_type=f32) + ue_b_ref[...])
        ce = jax.nn.relu(jnp.dot(ctx, ce_W_ref[...],
                                 preferred_element_type=f32) + ce_b_ref[...])
        fus_W = fus_W_ref[...]
        reg = (jnp.dot(ue, fus_W[:H], preferred_element_type=f32)
               + jnp.dot(ce, fus_W[H:], preferred_element_type=f32)
               + fus_b_ref[...])
        keep = jnp.logical_and(rowpos >= c0, rowpos < c01)
        out_ref[...] = jnp.where(keep, reg, out_ref[...])

    @pl.when(start < c0)
    def _():
        W_ih = W_ih_ref[...]
        W_hh = W_hh_ref[...]
        b_ih = b_ih_ref[...]
        b_hh = b_hh_ref[...]
        attn_W = attn_W_ref[...]
        attn_b = attn_b_ref[...]

        h = jnp.zeros((R, H), f32)
        s = jnp.zeros((R, 1), f32)
        acc = jnp.zeros((R, H), f32)
        for t in range(L):
            xt = xp[t]
            gi = jnp.dot(xt, W_ih, preferred_element_type=f32) + b_ih
            gh = jnp.dot(h, W_hh, preferred_element_type=f32) + b_hh
            rg = jax.nn.sigmoid(gi[:, :H] + gh[:, :H])
            z = jax.nn.sigmoid(gi[:, H:2 * H] + gh[:, H:2 * H])
            n = jnp.tanh(gi[:, 2 * H:] + rg * gh[:, 2 * H:])
            h = n + z * (h - n)
            sc = jnp.dot(h, attn_W, preferred_element_type=f32) + attn_b
            p = jnp.exp(sc)
            s = s + p
            acc = acc + p * h
        power = acc / s
        out_ref[...] = jnp.where(rowpos < c0, power, out_ref[...])


def _experts_call(perm, cnt, item, ctx_c, *weights):
    def full(shape):
        return pl.BlockSpec(shape, lambda b, *_: (0,) * len(shape))

    grid_spec = pltpu.PrefetchScalarGridSpec(
        num_scalar_prefetch=2,
        grid=(G,),
        in_specs=[
            pl.BlockSpec(memory_space=pltpu.MemorySpace.HBM),
            pl.BlockSpec((R, C), lambda b, *_: (b, 0)),
            full((D, 3 * H)), full((H, 3 * H)),
            full((1, 3 * H)), full((1, 3 * H)),
            full((H, 1)), full((1, 1)),
            full((D, H)), full((1, H)),
            full((C, H)), full((1, H)),
            full((2 * H, H)), full((1, H)),
            full((C, H)), full((1, H)),
            full((H, H)), full((1, H)),
        ],
        out_specs=pl.BlockSpec((R, H), lambda b, *_: (b, 0)),
        scratch_shapes=[
            pltpu.VMEM((2, L, R, D), jnp.float32),
            pltpu.SemaphoreType.DMA((2,)),
        ],
    )
    return pl.pallas_call(
        _experts_tc,
        grid_spec=grid_spec,
        out_shape=jax.ShapeDtypeStruct((B, H), jnp.float32),
    )(perm, cnt, item, ctx_c, *weights)


def kernel(user_segment, item_embeddings, context_embedding,
           W_ih, W_hh, b_ih, b_hh, attn_W, attn_b,
           ue_W, ue_b, ce_W, ce_b, fus_W, fus_b,
           cs_W1, cs_b1, cs_W2, cs_b2):
    seg = user_segment.astype(jnp.int32)

    perm, cnt = _route_sc(seg)
    ctx_c = _gather_ctx_sc(perm, context_embedding)
    out_c = _experts_call(
        perm, cnt, item_embeddings, ctx_c,
        W_ih, W_hh, b_ih.reshape(1, 3 * H), b_hh.reshape(1, 3 * H),
        attn_W, attn_b.reshape(1, 1),
        ue_W, ue_b.reshape(1, H), ce_W, ce_b.reshape(1, H),
        fus_W, fus_b.reshape(1, H),
        cs_W1, cs_b1.reshape(1, H), cs_W2, cs_b2.reshape(1, H))
    return _scatter_sc(perm, out_c)


# speedup vs baseline: 2.5612x; 1.0375x over previous
import functools

import jax
import jax.numpy as jnp
from jax import lax
from jax.experimental import pallas as pl
from jax.experimental.pallas import tpu as pltpu
from jax.experimental.pallas import tpu_sc as plsc

B, L, D, C, H = 4096, 50, 128, 128, 128
R = 256
G = B // R
NW = 32
RW = B // NW

_sc_mesh = plsc.VectorSubcoreMesh(core_axis_name="c", subcore_axis_name="s")
_sc_params = pltpu.CompilerParams(needs_layout_passes=False)


@functools.partial(
    pl.kernel,
    out_type=[jax.ShapeDtypeStruct((B,), jnp.int32),
              jax.ShapeDtypeStruct((16,), jnp.int32),
              jax.ShapeDtypeStruct((B, C), jnp.float32)],
    mesh=_sc_mesh,
    compiler_params=_sc_params,
    scratch_types=[pltpu.VMEM((B,), jnp.int32),
                   pltpu.VMEM((B,), jnp.int32),
                   pltpu.VMEM((16,), jnp.int32),
                   pltpu.VMEM((RW,), jnp.int32),
                   pltpu.VMEM((RW, C), jnp.float32),
                   pltpu.SemaphoreType.DMA],
)
def _route_sc(seg_hbm, ctx_hbm, perm_hbm, cnt_hbm, ctxc_hbm,
              seg_v, perm_v, cnt_v, idx_v, rows_v, sem):
    wid = lax.axis_index("s") * 2 + lax.axis_index("c")

    pltpu.sync_copy(seg_hbm, seg_v)
    lanes = lax.iota(jnp.int32, 16)
    one = jnp.ones((16,), jnp.int32)
    nil = jnp.zeros((16,), jnp.int32)

    def count_body(i, carry):
        c0, c1 = carry
        v = seg_v[pl.ds(i * 16, 16)]
        c0 = c0 + jnp.sum(jnp.where(v == 0, one, nil))
        c1 = c1 + jnp.sum(jnp.where(v == 1, one, nil))
        return c0, c1

    c0, c1 = lax.fori_loop(0, B // 16, count_body,
                           (jnp.int32(0), jnp.int32(0)))

    def scat_body(i, bases):
        b0, b1, b2 = bases
        v = seg_v[pl.ds(i * 16, 16)]
        rowid = lanes + i * 16
        m0 = v == 0
        m1 = v == 1
        m2 = v == 2
        p0 = plsc.cumsum(jnp.where(m0, one, nil))
        p1 = plsc.cumsum(jnp.where(m1, one, nil))
        p2 = plsc.cumsum(jnp.where(m2, one, nil))
        plsc.store_scatter(perm_v, [b0 + p0 - 1], rowid, mask=m0)
        plsc.store_scatter(perm_v, [b1 + p1 - 1], rowid, mask=m1)
        plsc.store_scatter(perm_v, [b2 + p2 - 1], rowid, mask=m2)
        return (b0 + jnp.sum(jnp.where(m0, one, nil)),
                b1 + jnp.sum(jnp.where(m1, one, nil)),
                b2 + jnp.sum(jnp.where(m2, one, nil)))

    lax.fori_loop(0, B // 16, scat_body, (jnp.int32(0), c0, c0 + c1))

    @pl.when(wid == 0)
    def _():
        cnt_v[...] = jnp.where(lanes == 0, c0,
                               jnp.where(lanes == 1, c0 + c1, nil))
        pltpu.sync_copy(perm_v, perm_hbm)
        pltpu.sync_copy(cnt_v, cnt_hbm)

    base = wid * RW

    def cp_body(j, carry):
        idx_v[pl.ds(j * 16, 16)] = perm_v[pl.ds(base + j * 16, 16)]
        return carry

    lax.fori_loop(0, RW // 16, cp_body, 0)
    pltpu.async_copy(ctx_hbm.at[idx_v], rows_v, sem).wait()
    pltpu.sync_copy(rows_v, ctxc_hbm.at[pl.ds(base, RW)])


@functools.partial(
    pl.kernel,
    out_type=jax.ShapeDtypeStruct((B, H), jnp.float32),
    mesh=_sc_mesh,
    compiler_params=_sc_params,
    scratch_types=[pltpu.VMEM((RW,), jnp.int32),
                   pltpu.VMEM((RW, H), jnp.float32),
                   pltpu.SemaphoreType.DMA],
)
def _scatter_sc(perm_hbm, outc_hbm, out_hbm, idx_v, rows_v, sem):
    wid = lax.axis_index("s") * 2 + lax.axis_index("c")
    base = wid * RW
    pltpu.sync_copy(perm_hbm.at[pl.ds(base, RW)], idx_v)
    pltpu.sync_copy(outc_hbm.at[pl.ds(base, RW)], rows_v)
    pltpu.async_copy(rows_v, out_hbm.at[idx_v], sem).wait()


def _experts_tc(perm_sref, cnt_sref, item_hbm, ctxc_ref,
                W_ih_ref, W_hh_ref, b_ih_ref, b_hh_ref,
                attn_W_ref, attn_b_ref,
                ue_W_ref, ue_b_ref, ce_W_ref, ce_b_ref,
                fus_W_ref, fus_b_ref,
                cs_W1_ref, cs_b1_ref, cs_W2_ref, cs_b2_ref,
                out_ref, xacc, sem):
    f32 = jnp.float32
    b = pl.program_id(0)
    start = b * R
    c0 = cnt_sref[0]
    c01 = cnt_sref[1]
    par = lax.rem(b, 2)

    def issue(blk, buf):
        s0 = blk * R

        def body(r, carry):
            idx = perm_sref[jnp.minimum(s0 + r, c01 - 1)]
            pltpu.make_async_copy(item_hbm.at[idx], xacc.at[buf, :, r, :],
                                  sem.at[buf]).start()
            return carry

        lax.fori_loop(0, R, body, 0, unroll=8)

    @pl.when(jnp.logical_and(b == 0, c01 > 0))
    def _():
        issue(0, 0)

    @pl.when(jnp.logical_and(b + 1 < G, (b + 1) * R < c01))
    def _():
        issue(b + 1, 1 - par)

    @pl.when(start < c01)
    def _():
        def dbody(r, carry):
            pltpu.make_async_copy(item_hbm.at[0], xacc.at[par, :, 0, :],
                                  sem.at[par]).wait()
            return carry

        lax.fori_loop(0, R, dbody, 0, unroll=8)

    xp = xacc.at[par]
    rowpos = start + lax.broadcasted_iota(jnp.int32, (R, 1), 0)
    ctx = ctxc_ref[...]

    cold = (jnp.dot(jax.nn.relu(jnp.dot(ctx, cs_W1_ref[...],
                                        preferred_element_type=f32)
                                + cs_b1_ref[...]),
                    cs_W2_ref[...], preferred_element_type=f32)
            + cs_b2_ref[...])
    out_ref[...] = cold

    @pl.when(jnp.logical_and(start < c01, start + R > c0))
    def _():
        xsum = jnp.zeros((R, D), f32)
        for t in range(L):
            xsum = xsum + xp[t]
        mean = xsum * (1.0 / L)
        ue = jax.nn.relu(jnp.dot(mean, ue_W_ref[...],
                                 preferred_element_type=f32) + ue_b_ref[...])
        ce = jax.nn.relu(jnp.dot(ctx, ce_W_ref[...],
                                 preferred_element_type=f32) + ce_b_ref[...])
        fus_W = fus_W_ref[...]
        reg = (jnp.dot(ue, fus_W[:H], preferred_element_type=f32)
               + jnp.dot(ce, fus_W[H:], preferred_element_type=f32)
               + fus_b_ref[...])
        keep = jnp.logical_and(rowpos >= c0, rowpos < c01)
        out_ref[...] = jnp.where(keep, reg, out_ref[...])

    @pl.when(start < c0)
    def _():
        W_ih = W_ih_ref[...]
        W_hh = W_hh_ref[...]
        b_ih = b_ih_ref[...]
        b_hh = b_hh_ref[...]
        attn_W = attn_W_ref[...]
        attn_b = attn_b_ref[...]

        h = jnp.zeros((R, H), f32)
        s = jnp.zeros((R, 1), f32)
        acc = jnp.zeros((R, H), f32)
        for t in range(L):
            xt = xp[t]
            gi = jnp.dot(xt, W_ih, preferred_element_type=f32) + b_ih
            gh = jnp.dot(h, W_hh, preferred_element_type=f32) + b_hh
            rg = jax.nn.sigmoid(gi[:, :H] + gh[:, :H])
            z = jax.nn.sigmoid(gi[:, H:2 * H] + gh[:, H:2 * H])
            n = jnp.tanh(gi[:, 2 * H:] + rg * gh[:, 2 * H:])
            h = n + z * (h - n)
            sc = jnp.dot(h, attn_W, preferred_element_type=f32) + attn_b
            p = jnp.exp(sc)
            s = s + p
            acc = acc + p * h
        power = acc / s
        out_ref[...] = jnp.where(rowpos < c0, power, out_ref[...])


def _experts_call(perm, cnt, item, ctx_c, *weights):
    def full(shape):
        return pl.BlockSpec(shape, lambda b, *_: (0,) * len(shape))

    grid_spec = pltpu.PrefetchScalarGridSpec(
        num_scalar_prefetch=2,
        grid=(G,),
        in_specs=[
            pl.BlockSpec(memory_space=pltpu.MemorySpace.HBM),
            pl.BlockSpec((R, C), lambda b, *_: (b, 0)),
            full((D, 3 * H)), full((H, 3 * H)),
            full((1, 3 * H)), full((1, 3 * H)),
            full((H, 1)), full((1, 1)),
            full((D, H)), full((1, H)),
            full((C, H)), full((1, H)),
            full((2 * H, H)), full((1, H)),
            full((C, H)), full((1, H)),
            full((H, H)), full((1, H)),
        ],
        out_specs=pl.BlockSpec((R, H), lambda b, *_: (b, 0)),
        scratch_shapes=[
            pltpu.VMEM((2, L, R, D), jnp.float32),
            pltpu.SemaphoreType.DMA((2,)),
        ],
    )
    return pl.pallas_call(
        _experts_tc,
        grid_spec=grid_spec,
        out_shape=jax.ShapeDtypeStruct((B, H), jnp.float32),
    )(perm, cnt, item, ctx_c, *weights)


def kernel(user_segment, item_embeddings, context_embedding,
           W_ih, W_hh, b_ih, b_hh, attn_W, attn_b,
           ue_W, ue_b, ce_W, ce_b, fus_W, fus_b,
           cs_W1, cs_b1, cs_W2, cs_b2):
    seg = user_segment.astype(jnp.int32)

    perm, cnt, ctx_c = _route_sc(seg, context_embedding)
    out_c = _experts_call(
        perm, cnt, item_embeddings, ctx_c,
        W_ih, W_hh, b_ih.reshape(1, 3 * H), b_hh.reshape(1, 3 * H),
        attn_W, attn_b.reshape(1, 1),
        ue_W, ue_b.reshape(1, H), ce_W, ce_b.reshape(1, H),
        fus_W, fus_b.reshape(1, H),
        cs_W1, cs_b1.reshape(1, H), cs_W2, cs_b2.reshape(1, H))
    return _scatter_sc(perm, out_c)


# speedup vs baseline: 2.5635x; 1.0009x over previous
import functools

import jax
import jax.numpy as jnp
from jax import lax
from jax.experimental import pallas as pl
from jax.experimental.pallas import tpu as pltpu
from jax.experimental.pallas import tpu_sc as plsc

B, L, D, C, H = 4096, 50, 128, 128, 128
R = 256
G = B // R
NW = 32
RW = B // NW

_sc_mesh = plsc.VectorSubcoreMesh(core_axis_name="c", subcore_axis_name="s")
_sc_params = pltpu.CompilerParams(needs_layout_passes=False)


@functools.partial(
    pl.kernel,
    out_type=[jax.ShapeDtypeStruct((B,), jnp.int32),
              jax.ShapeDtypeStruct((16,), jnp.int32),
              jax.ShapeDtypeStruct((B, C), jnp.float32)],
    mesh=_sc_mesh,
    compiler_params=_sc_params,
    scratch_types=[pltpu.VMEM((B,), jnp.int32),
                   pltpu.VMEM((B,), jnp.int32),
                   pltpu.VMEM((16,), jnp.int32),
                   pltpu.VMEM((RW,), jnp.int32),
                   pltpu.VMEM((RW, C), jnp.float32),
                   pltpu.SemaphoreType.DMA],
)
def _route_sc(seg_hbm, ctx_hbm, perm_hbm, cnt_hbm, ctxc_hbm,
              seg_v, perm_v, cnt_v, idx_v, rows_v, sem):
    wid = lax.axis_index("s") * 2 + lax.axis_index("c")

    pltpu.sync_copy(seg_hbm, seg_v)
    lanes = lax.iota(jnp.int32, 16)
    one = jnp.ones((16,), jnp.int32)
    nil = jnp.zeros((16,), jnp.int32)

    def count_body(i, carry):
        c0, c1 = carry
        v = seg_v[pl.ds(i * 16, 16)]
        c0 = c0 + jnp.sum(jnp.where(v == 0, one, nil))
        c1 = c1 + jnp.sum(jnp.where(v == 1, one, nil))
        return c0, c1

    c0, c1 = lax.fori_loop(0, B // 16, count_body,
                           (jnp.int32(0), jnp.int32(0)))

    def scat_body(i, bases):
        b0, b1, b2 = bases
        v = seg_v[pl.ds(i * 16, 16)]
        rowid = lanes + i * 16
        m0 = v == 0
        m1 = v == 1
        m2 = v == 2
        p0 = plsc.cumsum(jnp.where(m0, one, nil))
        p1 = plsc.cumsum(jnp.where(m1, one, nil))
        p2 = plsc.cumsum(jnp.where(m2, one, nil))
        plsc.store_scatter(perm_v, [b0 + p0 - 1], rowid, mask=m0)
        plsc.store_scatter(perm_v, [b1 + p1 - 1], rowid, mask=m1)
        plsc.store_scatter(perm_v, [b2 + p2 - 1], rowid, mask=m2)
        return (b0 + jnp.sum(jnp.where(m0, one, nil)),
                b1 + jnp.sum(jnp.where(m1, one, nil)),
                b2 + jnp.sum(jnp.where(m2, one, nil)))

    lax.fori_loop(0, B // 16, scat_body, (jnp.int32(0), c0, c0 + c1))

    @pl.when(wid == 0)
    def _():
        cnt_v[...] = jnp.where(lanes == 0, c0,
                               jnp.where(lanes == 1, c0 + c1, nil))
        pltpu.sync_copy(perm_v, perm_hbm)
        pltpu.sync_copy(cnt_v, cnt_hbm)

    base = wid * RW

    def cp_body(j, carry):
        idx_v[pl.ds(j * 16, 16)] = perm_v[pl.ds(base + j * 16, 16)]
        return carry

    lax.fori_loop(0, RW // 16, cp_body, 0)
    pltpu.async_copy(ctx_hbm.at[idx_v], rows_v, sem).wait()
    pltpu.sync_copy(rows_v, ctxc_hbm.at[pl.ds(base, RW)])


@functools.partial(
    pl.kernel,
    out_type=jax.ShapeDtypeStruct((B, H), jnp.float32),
    mesh=_sc_mesh,
    compiler_params=_sc_params,
    scratch_types=[pltpu.VMEM((RW,), jnp.int32),
                   pltpu.VMEM((RW, H), jnp.float32),
                   pltpu.SemaphoreType.DMA],
)
def _scatter_sc(perm_hbm, outc_hbm, out_hbm, idx_v, rows_v, sem):
    wid = lax.axis_index("s") * 2 + lax.axis_index("c")
    base = wid * RW
    pltpu.sync_copy(perm_hbm.at[pl.ds(base, RW)], idx_v)
    pltpu.sync_copy(outc_hbm.at[pl.ds(base, RW)], rows_v)
    pltpu.async_copy(rows_v, out_hbm.at[idx_v], sem).wait()


def _experts_tc(perm_sref, cnt_sref, item_hbm, ctxc_ref,
                W_ih_ref, W_hh_ref, b_ih_ref, b_hh_ref,
                attn_W_ref, attn_b_ref,
                ue_W_ref, ue_b_ref, ce_W_ref, ce_b_ref,
                fus_W_ref, fus_b_ref,
                cs_W1_ref, cs_b1_ref, cs_W2_ref, cs_b2_ref,
                out_ref, xacc, sem):
    f32 = jnp.float32
    b = pl.program_id(0)
    start = b * R
    c0 = cnt_sref[0]
    c01 = cnt_sref[1]
    par = lax.rem(b, 2)

    def issue(blk, buf):
        s0 = blk * R

        def body(r, carry):
            idx = perm_sref[jnp.minimum(s0 + r, c01 - 1)]
            pltpu.make_async_copy(item_hbm.at[idx], xacc.at[buf, :, r, :],
                                  sem.at[buf]).start()
            return carry

        lax.fori_loop(0, R, body, 0, unroll=8)

    @pl.when(jnp.logical_and(b == 0, c01 > 0))
    def _():
        issue(0, 0)

    @pl.when(jnp.logical_and(b + 1 < G, (b + 1) * R < c01))
    def _():
        issue(b + 1, 1 - par)

    @pl.when(start < c01)
    def _():
        def dbody(r, carry):
            pltpu.make_async_copy(item_hbm.at[0], xacc.at[par, :, 0, :],
                                  sem.at[par]).wait()
            return carry

        lax.fori_loop(0, R, dbody, 0, unroll=8)

    xp = xacc.at[par]
    rowpos = start + lax.broadcasted_iota(jnp.int32, (R, 1), 0)
    ctx = ctxc_ref[...]

    cold = (jnp.dot(jax.nn.relu(jnp.dot(ctx, cs_W1_ref[...],
                                        preferred_element_type=f32)
                                + cs_b1_ref[...]),
                    cs_W2_ref[...], preferred_element_type=f32)
            + cs_b2_ref[...])
    out_ref[...] = cold

    @pl.when(jnp.logical_and(start < c01, start + R > c0))
    def _():
        xsum = jnp.zeros((R, D), f32)
        for t in range(L):
            xsum = xsum + xp[t]
        mean = xsum * (1.0 / L)
        ue = jax.nn.relu(jnp.dot(mean, ue_W_ref[...],
                                 preferred_element_type=f32) + ue_b_ref[...])
        ce = jax.nn.relu(jnp.dot(ctx, ce_W_ref[...],
                                 preferred_element_type=f32) + ce_b_ref[...])
        fus_W = fus_W_ref[...]
        reg = (jnp.dot(ue, fus_W[:H], preferred_element_type=f32)
               + jnp.dot(ce, fus_W[H:], preferred_element_type=f32)
               + fus_b_ref[...])
        keep = jnp.logical_and(rowpos >= c0, rowpos < c01)
        out_ref[...] = jnp.where(keep, reg, out_ref[...])

    @pl.when(start < c0)
    def _():
        W_ih = W_ih_ref[...]
        W_hh = W_hh_ref[...]
        b_ih = b_ih_ref[...]
        b_hh = b_hh_ref[...]
        attn_W = attn_W_ref[...]
        attn_b = attn_b_ref[...]

        h = jnp.zeros((R, H), f32)
        s = jnp.zeros((R, 1), f32)
        acc = jnp.zeros((R, H), f32)
        bf16 = jnp.bfloat16
        for t in range(L):
            xt = xp[t]
            gi = jnp.dot(xt.astype(bf16), W_ih,
                         preferred_element_type=f32) + b_ih
            gh = jnp.dot(h.astype(bf16), W_hh,
                         preferred_element_type=f32) + b_hh
            rg = jax.nn.sigmoid(gi[:, :H] + gh[:, :H])
            z = jax.nn.sigmoid(gi[:, H:2 * H] + gh[:, H:2 * H])
            n = jnp.tanh(gi[:, 2 * H:] + rg * gh[:, 2 * H:])
            h = n + z * (h - n)
            sc = jnp.dot(h, attn_W, preferred_element_type=f32) + attn_b
            p = jnp.exp(sc)
            s = s + p
            acc = acc + p * h
        power = acc / s
        out_ref[...] = jnp.where(rowpos < c0, power, out_ref[...])


def _experts_call(perm, cnt, item, ctx_c, *weights):
    def full(shape):
        return pl.BlockSpec(shape, lambda b, *_: (0,) * len(shape))

    grid_spec = pltpu.PrefetchScalarGridSpec(
        num_scalar_prefetch=2,
        grid=(G,),
        in_specs=[
            pl.BlockSpec(memory_space=pltpu.MemorySpace.HBM),
            pl.BlockSpec((R, C), lambda b, *_: (b, 0)),
            full((D, 3 * H)), full((H, 3 * H)),
            full((1, 3 * H)), full((1, 3 * H)),
            full((H, 1)), full((1, 1)),
            full((D, H)), full((1, H)),
            full((C, H)), full((1, H)),
            full((2 * H, H)), full((1, H)),
            full((C, H)), full((1, H)),
            full((H, H)), full((1, H)),
        ],
        out_specs=pl.BlockSpec((R, H), lambda b, *_: (b, 0)),
        scratch_shapes=[
            pltpu.VMEM((2, L, R, D), jnp.float32),
            pltpu.SemaphoreType.DMA((2,)),
        ],
    )
    return pl.pallas_call(
        _experts_tc,
        grid_spec=grid_spec,
        out_shape=jax.ShapeDtypeStruct((B, H), jnp.float32),
    )(perm, cnt, item, ctx_c, *weights)


def kernel(user_segment, item_embeddings, context_embedding,
           W_ih, W_hh, b_ih, b_hh, attn_W, attn_b,
           ue_W, ue_b, ce_W, ce_b, fus_W, fus_b,
           cs_W1, cs_b1, cs_W2, cs_b2):
    seg = user_segment.astype(jnp.int32)

    perm, cnt, ctx_c = _route_sc(seg, context_embedding)
    out_c = _experts_call(
        perm, cnt, item_embeddings, ctx_c,
        W_ih.astype(jnp.bfloat16), W_hh.astype(jnp.bfloat16),
        b_ih.reshape(1, 3 * H), b_hh.reshape(1, 3 * H),
        attn_W, attn_b.reshape(1, 1),
        ue_W, ue_b.reshape(1, H), ce_W, ce_b.reshape(1, H),
        fus_W, fus_b.reshape(1, H),
        cs_W1, cs_b1.reshape(1, H), cs_W2, cs_b2.reshape(1, H))
    return _scatter_sc(perm, out_c)
